# Initial kernel scaffold; baseline (speedup 1.0000x reference)
import jax
import jax.numpy as jnp
from jax.experimental import pallas as pl


def kernel(x):
    raise NotImplementedError("write your pallas kernel here")


# speedup vs baseline: 2.9417x; 2.9417x over previous
import functools

import jax
import jax.numpy as jnp
from jax import lax
from jax.experimental import pallas as pl
from jax.experimental.pallas import tpu as pltpu
from jax.experimental.pallas import tpu_sc as plsc

N = 8192
D = 256
K = 1024
N_ITERS = 5
BLK = 512
NBLK = N // BLK

NUM_SC = 2
NUM_SUBCORES = 16
NUM_TILES = NUM_SC * NUM_SUBCORES
ROWS_PER_TILE = N // NUM_TILES
KROWS_PER_SUBCORE = K // NUM_SUBCORES
IDX_CHUNK = 128
NCHUNK = ROWS_PER_TILE // IDX_CHUNK


def _assign_body(x_ref, ct_ref, lab_ref, cnt_ref):
    i = pl.program_id(0)
    ct = ct_ref[...]
    c2 = jnp.sum(ct * ct, axis=0, keepdims=True)
    m = lax.dot_general(
        x_ref[...], ct, (((1,), (0,)), ((), ())),
        preferred_element_type=jnp.float32)
    score = c2 - 2.0 * m
    mn = jnp.min(score, axis=1, keepdims=True)
    cand = jnp.where(
        score == mn, lax.broadcasted_iota(jnp.int32, (BLK, K), 1), K)
    labels = jnp.min(cand, axis=1, keepdims=True)
    lab_ref[0, :, :] = labels
    onehot = (labels == lax.broadcasted_iota(jnp.int32, (BLK, K), 1))
    cnt = jnp.sum(onehot.astype(jnp.float32), axis=0, keepdims=True)

    @pl.when(i == 0)
    def _():
        cnt_ref[0, :, :] = jnp.zeros((1, K), jnp.float32)

    cnt_ref[0, :, :] += cnt


_assign_call = pl.pallas_call(
    _assign_body,
    grid=(NBLK,),
    in_specs=[
        pl.BlockSpec((BLK, D), lambda i: (i, 0)),
        pl.BlockSpec((D, K), lambda i: (0, 0)),
    ],
    out_specs=[
        pl.BlockSpec((1, BLK, 1), lambda i: (i, 0, 0)),
        pl.BlockSpec((1, 1, K), lambda i: (0, 0, 0)),
    ],
    out_shape=[
        jax.ShapeDtypeStruct((NBLK, BLK, 1), jnp.int32),
        jax.ShapeDtypeStruct((1, 1, K), jnp.float32),
    ],
)


def _assign(x, c):
    lab, cnt = _assign_call(x, c.T)
    return lab.reshape(N), cnt.reshape(K)


def _sc_segsum_body(x_hbm, lab_hbm, zeros_hbm, out_hbm, idx_a, idx_b, rows_v,
                    shared):
    cid = lax.axis_index("c")
    sid = lax.axis_index("s")
    wid = cid * NUM_SUBCORES + sid
    base = wid * ROWS_PER_TILE
    pltpu.sync_copy(
        zeros_hbm.at[pl.ds(sid * KROWS_PER_SUBCORE, KROWS_PER_SUBCORE)],
        shared.at[pl.ds(sid * KROWS_PER_SUBCORE, KROWS_PER_SUBCORE)])
    pltpu.sync_copy(x_hbm.at[pl.ds(base, ROWS_PER_TILE)], rows_v)
    pltpu.sync_copy(lab_hbm.at[wid * NCHUNK], idx_a)
    pltpu.sync_copy(lab_hbm.at[wid * NCHUNK + 1], idx_b)
    plsc.subcore_barrier()
    for j, idx in enumerate((idx_a, idx_b)):
        pltpu.sync_copy(
            rows_v.at[pl.ds(j * IDX_CHUNK, IDX_CHUNK)],
            shared.at[idx],
            add=True)
    plsc.subcore_barrier()
    pltpu.sync_copy(
        shared.at[pl.ds(sid * KROWS_PER_SUBCORE, KROWS_PER_SUBCORE)],
        out_hbm.at[cid].at[pl.ds(sid * KROWS_PER_SUBCORE, KROWS_PER_SUBCORE)])


@functools.cache
def _sc_segsum():
    mesh = plsc.VectorSubcoreMesh(core_axis_name="c", subcore_axis_name="s")
    return pl.kernel(
        _sc_segsum_body,
        mesh=mesh,
        compiler_params=pltpu.CompilerParams(use_tc_tiling_on_sc=False),
        out_type=jax.ShapeDtypeStruct((NUM_SC, K, D), jnp.float32),
        scratch_types=[
            pltpu.VMEM((IDX_CHUNK,), jnp.int32),
            pltpu.VMEM((IDX_CHUNK,), jnp.int32),
            pltpu.VMEM((ROWS_PER_TILE, D), jnp.float32),
            pltpu.VMEM_SHARED((K, D), jnp.float32),
        ],
    )


@jax.jit
def kernel(x):
    x = x.reshape(x.shape[0], -1)
    zeros = jnp.zeros((K, D), jnp.float32)
    c = x[:K]
    for _ in range(N_ITERS):
        labels, counts = _assign(x, c)
        partials = _sc_segsum()(x, labels.reshape(N // IDX_CHUNK, IDX_CHUNK), zeros)
        sums = partials[0] + partials[1]
        newc = sums / jnp.maximum(counts, 1.0)[:, None]
        c = jnp.where(counts[:, None] > 0, newc, c)
    return _assign(x, c)[0]
